# baseline (speedup 1.0000x reference)
import functools

import jax
import jax.numpy as jnp
from jax import lax
from jax.experimental import pallas as pl
from jax.experimental.pallas import tpu as pltpu
from jax.experimental.pallas import tpu_sc as plsc

_NB = 3
_CH = 16


def kernel(x, pos_embedding):
    B = x.shape[0]
    S, D = pos_embedding.shape
    info = plsc.get_sparse_core_info()
    NW = info.num_cores * info.num_subcores
    rows = S // NW
    nch = rows // _CH

    mesh = plsc.VectorSubcoreMesh(core_axis_name="c", subcore_axis_name="s")

    @functools.partial(
        pl.kernel,
        mesh=mesh,
        out_type=jax.ShapeDtypeStruct((B, S, D), pos_embedding.dtype),
        scratch_types=[
            pltpu.VMEM((_NB, _CH, D), pos_embedding.dtype),
            pltpu.SemaphoreType.DMA((_NB,)),
            pltpu.SemaphoreType.DMA((_NB,)),
        ],
    )
    def body(pos_hbm, out_hbm, buf, rsem, wsem):
        wid = lax.axis_index("s") * info.num_cores + lax.axis_index("c")
        base = wid * rows

        def read(i, slot):
            return pltpu.make_async_copy(
                pos_hbm.at[pl.ds(base + i * _CH, _CH), :],
                buf.at[slot],
                rsem.at[slot],
            )

        def write(slot, b, i):
            return pltpu.make_async_copy(
                buf.at[slot],
                out_hbm.at[b, pl.ds(base + i * _CH, _CH), :],
                wsem.at[slot],
            )

        read(0, 0).start()
        for i in range(nch):
            slot = i % _NB
            read(i, slot).wait()
            for b in range(B):
                write(slot, b, i).start()
            nxt = i + 1
            if nxt < nch:
                nslot = nxt % _NB
                if nxt >= _NB:
                    for b in range(B):
                        write(nslot, b, nxt - _NB).wait()
                read(nxt, nslot).start()
        for i in range(max(nch - _NB, 0), nch):
            slot = i % _NB
            for b in range(B):
                write(slot, b, i).wait()

    return body(pos_embedding)


# speedup vs baseline: 1.0601x; 1.0601x over previous
import functools

import jax
import jax.numpy as jnp
from jax import lax
from jax.experimental import pallas as pl
from jax.experimental.pallas import tpu as pltpu
from jax.experimental.pallas import tpu_sc as plsc

_NB = 3
_CH = 32


def kernel(x, pos_embedding):
    B = x.shape[0]
    S, D = pos_embedding.shape
    info = plsc.get_sparse_core_info()
    NW = info.num_cores * info.num_subcores
    rows = S // NW
    nch = rows // _CH

    mesh = plsc.VectorSubcoreMesh(core_axis_name="c", subcore_axis_name="s")

    @functools.partial(
        pl.kernel,
        mesh=mesh,
        out_type=jax.ShapeDtypeStruct((B, S, D), pos_embedding.dtype),
        scratch_types=[
            pltpu.VMEM((_NB, _CH, D), pos_embedding.dtype),
            pltpu.SemaphoreType.DMA((_NB,)),
            pltpu.SemaphoreType.DMA((_NB,)),
        ],
    )
    def body(pos_hbm, out_hbm, buf, rsem, wsem):
        wid = lax.axis_index("s") * info.num_cores + lax.axis_index("c")
        base = wid * rows

        def read(i, slot):
            return pltpu.make_async_copy(
                pos_hbm.at[pl.ds(base + i * _CH, _CH), :],
                buf.at[slot],
                rsem.at[slot],
            )

        def write(slot, b, i):
            return pltpu.make_async_copy(
                buf.at[slot],
                out_hbm.at[b, pl.ds(base + i * _CH, _CH), :],
                wsem.at[slot],
            )

        read(0, 0).start()
        for i in range(nch):
            slot = i % _NB
            read(i, slot).wait()
            for b in range(B):
                write(slot, b, i).start()
            nxt = i + 1
            if nxt < nch:
                nslot = nxt % _NB
                if nxt >= _NB:
                    for b in range(B):
                        write(nslot, b, nxt - _NB).wait()
                read(nxt, nslot).start()
        for i in range(max(nch - _NB, 0), nch):
            slot = i % _NB
            for b in range(B):
                write(slot, b, i).wait()

    return body(pos_embedding)


# speedup vs baseline: 1.0930x; 1.0310x over previous
import functools

import jax
import jax.numpy as jnp
from jax import lax
from jax.experimental import pallas as pl
from jax.experimental.pallas import tpu as pltpu
from jax.experimental.pallas import tpu_sc as plsc

_NB = 2
_CH = 56


def kernel(x, pos_embedding):
    B = x.shape[0]
    S, D = pos_embedding.shape
    info = plsc.get_sparse_core_info()
    NW = info.num_cores * info.num_subcores
    rows = S // NW
    chunks = []
    off = 0
    while off < rows:
        sz = min(_CH, rows - off)
        chunks.append((off, sz))
        off += sz
    nch = len(chunks)

    mesh = plsc.VectorSubcoreMesh(core_axis_name="c", subcore_axis_name="s")

    @functools.partial(
        pl.kernel,
        mesh=mesh,
        out_type=jax.ShapeDtypeStruct((B, S, D), pos_embedding.dtype),
        scratch_types=[
            pltpu.VMEM((_NB, _CH, D), pos_embedding.dtype),
            pltpu.SemaphoreType.DMA((_NB,)),
            pltpu.SemaphoreType.DMA((_NB,)),
        ],
    )
    def body(pos_hbm, out_hbm, buf, rsem, wsem):
        wid = lax.axis_index("s") * info.num_cores + lax.axis_index("c")
        base = wid * rows

        def read(i, slot):
            off, sz = chunks[i]
            return pltpu.make_async_copy(
                pos_hbm.at[pl.ds(base + off, sz), :],
                buf.at[slot, pl.ds(0, sz)],
                rsem.at[slot],
            )

        def write(slot, b, i):
            off, sz = chunks[i]
            return pltpu.make_async_copy(
                buf.at[slot, pl.ds(0, sz)],
                out_hbm.at[b, pl.ds(base + off, sz), :],
                wsem.at[slot],
            )

        read(0, 0).start()
        for i in range(nch):
            slot = i % _NB
            read(i, slot).wait()
            for b in range(B):
                write(slot, b, i).start()
            nxt = i + 1
            if nxt < nch:
                nslot = nxt % _NB
                if nxt >= _NB:
                    for b in range(B):
                        write(nslot, b, nxt - _NB).wait()
                read(nxt, nslot).start()
        for i in range(max(nch - _NB, 0), nch):
            slot = i % _NB
            for b in range(B):
                write(slot, b, i).wait()

    return body(pos_embedding)


# speedup vs baseline: 1.1014x; 1.0078x over previous
import functools

import jax
import jax.numpy as jnp
from jax import lax
from jax.experimental import pallas as pl
from jax.experimental.pallas import tpu as pltpu
from jax.experimental.pallas import tpu_sc as plsc

_NB = 2
_CH = 56


def kernel(x, pos_embedding):
    B = x.shape[0]
    S, D = pos_embedding.shape
    info = plsc.get_sparse_core_info()
    NW = info.num_cores * info.num_subcores
    rows = S // NW
    caps = (64, 56)
    chunks = []
    off = 0
    while off < rows:
        sz = min(caps[len(chunks) % _NB], rows - off)
        chunks.append((off, sz))
        off += sz
    nch = len(chunks)

    mesh = plsc.VectorSubcoreMesh(core_axis_name="c", subcore_axis_name="s")

    @functools.partial(
        pl.kernel,
        mesh=mesh,
        out_type=jax.ShapeDtypeStruct((B, S, D), pos_embedding.dtype),
        scratch_types=[
            pltpu.VMEM((caps[0], D), pos_embedding.dtype),
            pltpu.VMEM((caps[1], D), pos_embedding.dtype),
            pltpu.SemaphoreType.DMA((_NB,)),
            pltpu.SemaphoreType.DMA((_NB,)),
        ],
    )
    def body(pos_hbm, out_hbm, buf_a, buf_b, rsem, wsem):
        wid = lax.axis_index("s") * info.num_cores + lax.axis_index("c")
        base = wid * rows
        bufs = (buf_a, buf_b)

        def read(i, slot):
            off, sz = chunks[i]
            return pltpu.make_async_copy(
                pos_hbm.at[pl.ds(base + off, sz), :],
                bufs[slot].at[pl.ds(0, sz)],
                rsem.at[slot],
            )

        def write(slot, b, i):
            off, sz = chunks[i]
            return pltpu.make_async_copy(
                bufs[slot].at[pl.ds(0, sz)],
                out_hbm.at[b, pl.ds(base + off, sz), :],
                wsem.at[slot],
            )

        read(0, 0).start()
        for i in range(nch):
            slot = i % _NB
            read(i, slot).wait()
            for b in range(B):
                write(slot, b, i).start()
            nxt = i + 1
            if nxt < nch:
                nslot = nxt % _NB
                if nxt >= _NB:
                    for b in range(B):
                        write(nslot, b, nxt - _NB).wait()
                read(nxt, nslot).start()
        for i in range(max(nch - _NB, 0), nch):
            slot = i % _NB
            for b in range(B):
                write(slot, b, i).wait()

    return body(pos_embedding)
